# Initial kernel scaffold; baseline (speedup 1.0000x reference)
import jax
import jax.numpy as jnp
from jax import lax
from jax.experimental import pallas as pl
from jax.experimental.pallas import tpu as pltpu
from jax.experimental.pallas import tpu_sc as plsc

B = 128
N = 32768
K = 16
L = 16
NC = 2
NS = 16
NW = NC * NS
ROWS_PER_W = B // NW

NEG = jnp.float32(-jnp.inf)
BIGI = jnp.int32(2**30)
CAND_CAP = 1024

_PASS_A_ACCS = 8
_PASS_B_GRP = 4


def _topk_body(scores_hbm, out_hbm, row_v, outrow_v, cvals, cidxs):
    wid = lax.axis_index("c") * NS + lax.axis_index("s")
    lane = lax.iota(jnp.int32, L)

    @plsc.parallel_loop(0, N // L)
    def _zero(i):
        outrow_v[pl.ds(i * L, L)] = jnp.zeros((L,), jnp.float32)

    for rr in range(ROWS_PER_W):
        row = wid * ROWS_PER_W + rr
        pltpu.sync_copy(scores_hbm.at[row], row_v)

        accs0 = tuple(jnp.full((L,), NEG) for _ in range(_PASS_A_ACCS))

        @plsc.parallel_loop(0, N // (L * _PASS_A_ACCS), carry=accs0)
        def _pass_a(i, accs):
            base = i * (L * _PASS_A_ACCS)
            return tuple(
                jnp.maximum(a, row_v[pl.ds(base + k * L, L)])
                for k, a in enumerate(accs)
            )

        m = _pass_a[0]
        for a in _pass_a[1:]:
            m = jnp.maximum(m, a)
        t0 = jnp.min(m)

        grp = L * _PASS_B_GRP

        @plsc.parallel_loop(0, N // grp, carry=jnp.int32(0))
        def _pass_b(i, off):
            base = i * grp
            vs = [row_v[pl.ds(base + k * L, L)] for k in range(_PASS_B_GRP)]
            ms = [v >= t0 for v in vs]
            anym = ms[0]
            for mk in ms[1:]:
                anym = anym | mk

            def slow(off):
                for k in range(_PASS_B_GRP):
                    cnt = jnp.max(plsc.all_reduce_population_count(ms[k]))
                    plsc.store_compressed(
                        cvals.at[pl.ds(off, L)], vs[k], mask=ms[k])
                    plsc.store_compressed(
                        cidxs.at[pl.ds(off, L)],
                        lane + (base + k * L), mask=ms[k])
                    off = jnp.minimum(off + cnt, CAND_CAP)
                return off

            return lax.cond(jnp.any(anym), slow, lambda o: o, off)

        ncand = _pass_b
        nv = (ncand + (L - 1)) // L

        def _clean(j, _):
            pos = lane + j * L
            v = cvals[pl.ds(j * L, L)]
            cvals[pl.ds(j * L, L)] = jnp.where(pos < ncand, v, NEG)
            return 0

        lax.fori_loop(nv - 1, nv, _clean, 0)

        def _round(r, selvec):
            def scan(j, st):
                bv, bi = st
                v = cvals[pl.ds(j * L, L)]
                ix = cidxs[pl.ds(j * L, L)]
                take = (v > bv) | ((v == bv) & (ix < bi))
                return (jnp.where(take, v, bv), jnp.where(take, ix, bi))

            bv, bi = lax.fori_loop(
                0, nv, scan,
                (jnp.full((L,), NEG), jnp.full((L,), BIGI)))
            mval = jnp.max(bv)
            sel = jnp.min(jnp.where(bv == mval, bi, BIGI))

            def suppress(j, _):
                v = cvals[pl.ds(j * L, L)]
                ix = cidxs[pl.ds(j * L, L)]
                cvals[pl.ds(j * L, L)] = jnp.where(ix == sel, NEG, v)
                return 0

            lax.fori_loop(0, nv, suppress, 0)
            return jnp.where(lane == r, sel, selvec)

        selvec = lax.fori_loop(0, K, _round, jnp.full((L,), BIGI))

        plsc.store_scatter(outrow_v, [selvec], jnp.ones((L,), jnp.float32))
        pltpu.sync_copy(outrow_v, out_hbm.at[row])
        plsc.store_scatter(outrow_v, [selvec], jnp.zeros((L,), jnp.float32))


@jax.jit
def _topk_mask(scores):
    mesh = plsc.VectorSubcoreMesh(
        core_axis_name="c", subcore_axis_name="s")
    return pl.kernel(
        _topk_body,
        out_type=jax.ShapeDtypeStruct((B, N), jnp.float32),
        mesh=mesh,
        scratch_types=[
            pltpu.VMEM((N,), jnp.float32),
            pltpu.VMEM((N,), jnp.float32),
            pltpu.VMEM((CAND_CAP + L,), jnp.float32),
            pltpu.VMEM((CAND_CAP + L,), jnp.int32),
        ],
    )(scores)


def kernel(scores):
    return _topk_mask(scores)


# speedup vs baseline: 8.3787x; 8.3787x over previous
import jax
import jax.numpy as jnp
from jax import lax
from jax.experimental import pallas as pl
from jax.experimental.pallas import tpu as pltpu
from jax.experimental.pallas import tpu_sc as plsc

B = 128
N = 32768
K = 16
L = 16
NC = 2
NS = 16
NW = NC * NS
ROWS_PER_W = B // NW

NEG = float("-inf")
BIGI = 2**30
CAND_CAP = 1024

_PASS_A_ACCS = 8
_PASS_B_GRP = 4


def _vmax_scalar(x):
    return plsc.cummax(x)[L - 1]


def _vmin_scalar(x):
    return -plsc.cummax(-x)[L - 1]


def _topk_body(scores_hbm, out_hbm, row_v, outrow_v, cvals, cidxs):
    wid = lax.axis_index("c") * NS + lax.axis_index("s")
    lane = lax.iota(jnp.int32, L)

    @plsc.parallel_loop(0, N // L)
    def _zero(i):
        outrow_v[pl.ds(i * L, L)] = jnp.zeros((L,), jnp.float32)

    for rr in range(ROWS_PER_W):
        row = wid * ROWS_PER_W + rr
        pltpu.sync_copy(scores_hbm.at[row], row_v)

        accs0 = tuple(jnp.full((L,), NEG) for _ in range(_PASS_A_ACCS))

        @plsc.parallel_loop(0, N // (L * _PASS_A_ACCS), carry=accs0)
        def _pass_a(i, accs):
            base = i * (L * _PASS_A_ACCS)
            return tuple(
                jnp.maximum(a, row_v[pl.ds(base + k * L, L)])
                for k, a in enumerate(accs)
            )

        m = _pass_a[0]
        for a in _pass_a[1:]:
            m = jnp.maximum(m, a)
        t0 = _vmin_scalar(m)

        grp = L * _PASS_B_GRP

        @plsc.parallel_loop(0, N // grp, carry=jnp.int32(0))
        def _pass_b(i, off):
            base = i * grp
            vs = [row_v[pl.ds(base + k * L, L)] for k in range(_PASS_B_GRP)]
            ms = [v >= t0 for v in vs]
            anym = ms[0]
            for mk in ms[1:]:
                anym = anym | mk

            def slow(off):
                for k in range(_PASS_B_GRP):
                    cnt = plsc.all_reduce_population_count(ms[k])[0]
                    plsc.store_compressed(
                        cvals.at[pl.ds(off, L)], vs[k], mask=ms[k])
                    plsc.store_compressed(
                        cidxs.at[pl.ds(off, L)],
                        lane + (base + k * L), mask=ms[k])
                    off = jnp.minimum(off + cnt, CAND_CAP)
                return off

            have = plsc.all_reduce_population_count(anym)[0] > 0
            return lax.cond(have, slow, lambda o: o, off)

        ncand = _pass_b
        nv = (ncand + (L - 1)) // L

        def _clean(j, _):
            pos = lane + j * L
            v = cvals[pl.ds(j * L, L)]
            cvals[pl.ds(j * L, L)] = jnp.where(pos < ncand, v, NEG)
            return 0

        lax.fori_loop(nv - 1, nv, _clean, 0)

        def _round(r, selvec):
            def scan(j, st):
                bv, bi = st
                v = cvals[pl.ds(j * L, L)]
                ix = cidxs[pl.ds(j * L, L)]
                take = (v > bv) | ((v == bv) & (ix < bi))
                return (jnp.where(take, v, bv), jnp.where(take, ix, bi))

            bv, bi = lax.fori_loop(
                0, nv, scan,
                (jnp.full((L,), NEG), jnp.full((L,), BIGI)))
            mval = _vmax_scalar(bv)
            sel = _vmin_scalar(jnp.where(bv == mval, bi, BIGI))

            def suppress(j, _):
                v = cvals[pl.ds(j * L, L)]
                ix = cidxs[pl.ds(j * L, L)]
                cvals[pl.ds(j * L, L)] = jnp.where(ix == sel, NEG, v)
                return 0

            lax.fori_loop(0, nv, suppress, 0)
            return jnp.where(lane == r, sel, selvec)

        selvec = lax.fori_loop(0, K, _round, jnp.full((L,), BIGI))

        plsc.store_scatter(outrow_v, [selvec], jnp.ones((L,), jnp.float32))
        pltpu.sync_copy(outrow_v, out_hbm.at[row])
        plsc.store_scatter(outrow_v, [selvec], jnp.zeros((L,), jnp.float32))


@jax.jit
def _topk_mask(scores):
    mesh = plsc.VectorSubcoreMesh(
        core_axis_name="c", subcore_axis_name="s")
    return pl.kernel(
        _topk_body,
        out_type=jax.ShapeDtypeStruct((B, N), jnp.float32),
        mesh=mesh,
        compiler_params=pltpu.CompilerParams(needs_layout_passes=False),
        scratch_types=[
            pltpu.VMEM((N,), jnp.float32),
            pltpu.VMEM((N,), jnp.float32),
            pltpu.VMEM((CAND_CAP + L,), jnp.float32),
            pltpu.VMEM((CAND_CAP + L,), jnp.int32),
        ],
    )(scores)


def kernel(scores):
    return _topk_mask(scores)
